# baseline (speedup 1.0000x reference)
import jax
import jax.numpy as jnp
from jax import lax
from jax.experimental import pallas as pl
from jax.experimental.pallas import tpu as pltpu
from jax.experimental.pallas import tpu_sc as plsc

_NC = 2
_NS = 16
_L = 16

_POOL = 4
_D = 1024
_N_OUT = 2048

_SC_OUT = 512
_ROWS_PER_W = _SC_OUT // (_NC * _NS)
_CHUNK_OUT = 4
_N_CHUNKS = _ROWS_PER_W // _CHUNK_OUT
_CHUNK_IN = _CHUNK_OUT * _POOL


def _sc_body(x_hbm, out_hbm, in_v0, in_v1, out_v0, out_v1,
             isem0, isem1, osem0, osem1):
    wid = lax.axis_index("s") * _NC + lax.axis_index("c")
    out_base = wid * _ROWS_PER_W
    in_base = out_base * _POOL

    in_bufs = (in_v0, in_v1)
    out_bufs = (out_v0, out_v1)
    in_sems = (isem0, isem1)
    out_sems = (osem0, osem1)

    def start_in(c):
        return pltpu.async_copy(
            x_hbm.at[pl.ds(in_base + c * _CHUNK_IN, _CHUNK_IN)],
            in_bufs[c % 2], in_sems[c % 2])

    def start_out(c):
        return pltpu.async_copy(
            out_bufs[c % 2],
            out_hbm.at[pl.ds(out_base + c * _CHUNK_OUT, _CHUNK_OUT)],
            out_sems[c % 2])

    pending_in = {0: start_in(0)}
    pending_out = {}

    for c in range(_N_CHUNKS):
        if c + 1 < _N_CHUNKS:
            pending_in[c + 1] = start_in(c + 1)
        pending_in.pop(c).wait()
        if c - 2 in pending_out:
            pending_out.pop(c - 2).wait()

        in_v = in_bufs[c % 2]
        out_v = out_bufs[c % 2]

        @plsc.parallel_loop(0, _D // _L, unroll=8)
        def col_group(g):
            off = pl.multiple_of(g * _L, _L)
            for r in range(_CHUNK_OUT):
                a = in_v[4 * r + 0, pl.ds(off, _L)]
                b = in_v[4 * r + 1, pl.ds(off, _L)]
                cc = in_v[4 * r + 2, pl.ds(off, _L)]
                d = in_v[4 * r + 3, pl.ds(off, _L)]
                out_v[r, pl.ds(off, _L)] = ((a + b) + (cc + d)) * 0.25

        pending_out[c] = start_out(c)

    for c in sorted(pending_out):
        pending_out.pop(c).wait()


def _sc_pool(x):
    mesh = plsc.VectorSubcoreMesh(core_axis_name="c", subcore_axis_name="s")
    return pl.kernel(
        _sc_body,
        out_type=jax.ShapeDtypeStruct((_SC_OUT, _D), jnp.float32),
        mesh=mesh,
        scratch_types=[
            pltpu.VMEM((_CHUNK_IN, _D), jnp.float32),
            pltpu.VMEM((_CHUNK_IN, _D), jnp.float32),
            pltpu.VMEM((_CHUNK_OUT, _D), jnp.float32),
            pltpu.VMEM((_CHUNK_OUT, _D), jnp.float32),
            pltpu.SemaphoreType.DMA,
            pltpu.SemaphoreType.DMA,
            pltpu.SemaphoreType.DMA,
            pltpu.SemaphoreType.DMA,
        ],
    )(x)


def _tc_body(x_ref, o_ref):
    s = (x_ref[0::4, :] + x_ref[1::4, :]) + (x_ref[2::4, :] + x_ref[3::4, :])
    o_ref[...] = s * 0.25


def _tc_body_aliased(x_ref, prev_ref, o_ref):
    del prev_ref
    s = (x_ref[0::4, :] + x_ref[1::4, :]) + (x_ref[2::4, :] + x_ref[3::4, :])
    o_ref[...] = s * 0.25


def _tc_pool(x):
    b_out = pl.pallas_call(
        _tc_body,
        grid=(1, _D // 128),
        in_specs=[pl.BlockSpec((2048, 128), lambda i, j: (i + 1, j))],
        out_specs=pl.BlockSpec((512, 128), lambda i, j: (i + 1, j)),
        out_shape=jax.ShapeDtypeStruct((_N_OUT, _D), jnp.float32),
    )(x)
    return pl.pallas_call(
        _tc_body_aliased,
        grid=(1, _D // 128),
        in_specs=[
            pl.BlockSpec((4096, 128), lambda i, j: (i + 1, j)),
            pl.BlockSpec(memory_space=pl.ANY),
        ],
        out_specs=pl.BlockSpec((1024, 128), lambda i, j: (i + 1, j)),
        out_shape=jax.ShapeDtypeStruct((_N_OUT, _D), jnp.float32),
        input_output_aliases={1: 0},
    )(x, b_out)


def kernel(x, comp_rate, seqlens):
    del comp_rate, seqlens
    sc_out = _sc_pool(x)
    tc_out = _tc_pool(x)
    return lax.dynamic_update_slice(tc_out, sc_out, (0, 0))


# speedup vs baseline: 1.0226x; 1.0226x over previous
import jax
import jax.numpy as jnp
from jax import lax
from jax.experimental import pallas as pl
from jax.experimental.pallas import tpu as pltpu
from jax.experimental.pallas import tpu_sc as plsc

_NC = 2
_NS = 16
_L = 16

_POOL = 4
_D = 1024
_N_OUT = 2048

_SC_OUT = 512
_ROWS_PER_W = _SC_OUT // (_NC * _NS)
_CHUNK_OUT = 8
_N_CHUNKS = _ROWS_PER_W // _CHUNK_OUT
_CHUNK_IN = _CHUNK_OUT * _POOL


def _sc_body(x_hbm, out_hbm, in_v0, in_v1, out_v0, out_v1,
             isem0, isem1, osem0, osem1):
    wid = lax.axis_index("s") * _NC + lax.axis_index("c")
    out_base = wid * _ROWS_PER_W
    in_base = out_base * _POOL

    in_bufs = (in_v0, in_v1)
    out_bufs = (out_v0, out_v1)
    in_sems = (isem0, isem1)
    out_sems = (osem0, osem1)

    def start_in(c):
        return pltpu.async_copy(
            x_hbm.at[pl.ds(in_base + c * _CHUNK_IN, _CHUNK_IN)],
            in_bufs[c % 2], in_sems[c % 2])

    def start_out(c):
        return pltpu.async_copy(
            out_bufs[c % 2],
            out_hbm.at[pl.ds(out_base + c * _CHUNK_OUT, _CHUNK_OUT)],
            out_sems[c % 2])

    pending_in = {0: start_in(0)}
    pending_out = {}

    for c in range(_N_CHUNKS):
        if c + 1 < _N_CHUNKS:
            pending_in[c + 1] = start_in(c + 1)
        pending_in.pop(c).wait()
        if c - 2 in pending_out:
            pending_out.pop(c - 2).wait()

        in_v = in_bufs[c % 2]
        out_v = out_bufs[c % 2]

        @plsc.parallel_loop(0, _D // _L, unroll=4)
        def col_group(g):
            off = pl.multiple_of(g * _L, _L)
            for r in range(_CHUNK_OUT):
                a = in_v[4 * r + 0, pl.ds(off, _L)]
                b = in_v[4 * r + 1, pl.ds(off, _L)]
                cc = in_v[4 * r + 2, pl.ds(off, _L)]
                d = in_v[4 * r + 3, pl.ds(off, _L)]
                out_v[r, pl.ds(off, _L)] = ((a + b) + (cc + d)) * 0.25

        pending_out[c] = start_out(c)

    for c in sorted(pending_out):
        pending_out.pop(c).wait()


def _sc_pool(x):
    mesh = plsc.VectorSubcoreMesh(core_axis_name="c", subcore_axis_name="s")
    return pl.kernel(
        _sc_body,
        out_type=jax.ShapeDtypeStruct((_SC_OUT, _D), jnp.float32),
        mesh=mesh,
        scratch_types=[
            pltpu.VMEM((_CHUNK_IN, _D), jnp.float32),
            pltpu.VMEM((_CHUNK_IN, _D), jnp.float32),
            pltpu.VMEM((_CHUNK_OUT, _D), jnp.float32),
            pltpu.VMEM((_CHUNK_OUT, _D), jnp.float32),
            pltpu.SemaphoreType.DMA,
            pltpu.SemaphoreType.DMA,
            pltpu.SemaphoreType.DMA,
            pltpu.SemaphoreType.DMA,
        ],
    )(x)


def _tc_body(x_ref, o_ref):
    s = (x_ref[0::4, :] + x_ref[1::4, :]) + (x_ref[2::4, :] + x_ref[3::4, :])
    o_ref[...] = s * 0.25


def _tc_body_aliased(x_ref, prev_ref, o_ref):
    del prev_ref
    s = (x_ref[0::4, :] + x_ref[1::4, :]) + (x_ref[2::4, :] + x_ref[3::4, :])
    o_ref[...] = s * 0.25


def _tc_pool(x):
    b_out = pl.pallas_call(
        _tc_body,
        grid=(1, _D // 128),
        in_specs=[pl.BlockSpec((2048, 128), lambda i, j: (i + 1, j))],
        out_specs=pl.BlockSpec((512, 128), lambda i, j: (i + 1, j)),
        out_shape=jax.ShapeDtypeStruct((_N_OUT, _D), jnp.float32),
    )(x)
    return pl.pallas_call(
        _tc_body_aliased,
        grid=(1, _D // 128),
        in_specs=[
            pl.BlockSpec((4096, 128), lambda i, j: (i + 1, j)),
            pl.BlockSpec(memory_space=pl.ANY),
        ],
        out_specs=pl.BlockSpec((1024, 128), lambda i, j: (i + 1, j)),
        out_shape=jax.ShapeDtypeStruct((_N_OUT, _D), jnp.float32),
        input_output_aliases={1: 0},
    )(x, b_out)


def kernel(x, comp_rate, seqlens):
    del comp_rate, seqlens
    sc_out = _sc_pool(x)
    tc_out = _tc_pool(x)
    return lax.dynamic_update_slice(tc_out, sc_out, (0, 0))
